# Initial kernel scaffold; baseline (speedup 1.0000x reference)
import jax
import jax.numpy as jnp
from jax.experimental import pallas as pl


def kernel(x, edge_index, W1, b1, W2, b2):
    raise NotImplementedError("write your pallas kernel here")


# speedup vs baseline: 12.2455x; 12.2455x over previous
import jax
import jax.numpy as jnp
from jax import lax
from jax.experimental import pallas as pl
from jax.experimental.pallas import tpu as pltpu
from jax.experimental.pallas import tpu_sc as plsc

N_NODES = 10000
N_EDGES = 320000
D = 128

NUM_CORES = 2
NUM_SUBCORES = 16
NW = NUM_CORES * NUM_SUBCORES
EPT = N_EDGES // NW
CHUNK = 50
NBLK = 5
BLKC = EPT // (NBLK * CHUNK)
NBUF = 5
NPAD = 10240
ROWS_PER_TILE = NPAD // NUM_SUBCORES


def _make_aggregate(with_count):
  out_type = [jax.ShapeDtypeStruct((NUM_CORES, NPAD, D), jnp.float32)]
  scratch = [
      pltpu.VMEM((BLKC, CHUNK), jnp.int32),
      pltpu.VMEM((BLKC, CHUNK), jnp.int32),
      [pltpu.VMEM((CHUNK, D), jnp.float32) for _ in range(NBUF)],
      pltpu.VMEM_SHARED((NPAD, D), jnp.float32),
      [pltpu.SemaphoreType.DMA for _ in range(NBUF)],
  ]
  if with_count:
    out_type.append(
        jax.ShapeDtypeStruct((NUM_CORES, NPAD, 16), jnp.float32))
    scratch += [
        pltpu.VMEM((CHUNK, 16), jnp.float32),
        pltpu.VMEM_SHARED((NPAD, 16), jnp.float32),
        pltpu.SemaphoreType.DMA,
    ]

  def body(feat_h, src_h, dst_h, zrow_h, zcnt_h, ones_h, *rest):
    if with_count:
      (s_out, c_out, sidx, didx, rows, acc, sems, ones, cacc, csem) = rest
    else:
      (s_out, sidx, didx, rows, acc, sems) = rest

    cid = lax.axis_index("c")
    sid = lax.axis_index("s")
    wid = cid * NUM_SUBCORES + sid
    row0 = sid * ROWS_PER_TILE

    pltpu.sync_copy(zrow_h, acc.at[pl.ds(row0, ROWS_PER_TILE)])
    if with_count:
      pltpu.sync_copy(zcnt_h, cacc.at[pl.ds(row0, ROWS_PER_TILE)])
      pltpu.sync_copy(ones_h, ones)

    plsc.subcore_barrier()

    def block(blk, _):
      pltpu.sync_copy(src_h.at[wid, blk], sidx)
      pltpu.sync_copy(dst_h.at[wid, blk], didx)
      for b in range(NBUF):
        pltpu.async_copy(feat_h.at[sidx.at[b]], rows[b], sems[b])

      def ring(q, _):
        for b in range(NBUF):
          j = NBUF * q + b
          pltpu.make_async_copy(feat_h.at[sidx.at[j]], rows[b],
                                sems[b]).wait()
          pltpu.sync_copy(rows[b], acc.at[didx.at[j]], add=True)
          if with_count:
            pltpu.async_copy(ones, cacc.at[didx.at[j]], csem, add=True)

          @pl.when(q < BLKC // NBUF - 1)
          def _():
            pltpu.async_copy(feat_h.at[sidx.at[j + NBUF]], rows[b], sems[b])
        return 0
      lax.fori_loop(0, BLKC // NBUF, ring, 0)
      if with_count:
        def cdrain(j, _):
          pltpu.make_async_copy(ones, cacc.at[didx.at[j]], csem).wait()
          return 0
        lax.fori_loop(0, BLKC, cdrain, 0)
      return 0
    lax.fori_loop(0, NBLK, block, 0)

    plsc.subcore_barrier()

    pltpu.sync_copy(acc.at[pl.ds(row0, ROWS_PER_TILE)],
                    s_out.at[cid, pl.ds(row0, ROWS_PER_TILE)])
    if with_count:
      pltpu.sync_copy(cacc.at[pl.ds(row0, ROWS_PER_TILE)],
                      c_out.at[cid, pl.ds(row0, ROWS_PER_TILE)])

  mesh = plsc.VectorSubcoreMesh(core_axis_name="c", subcore_axis_name="s")
  return pl.kernel(body, out_type=tuple(out_type), mesh=mesh,
                   scratch_types=scratch,
                   compiler_params=pltpu.CompilerParams(
                       use_tc_tiling_on_sc=False))


_aggregate_count = _make_aggregate(True)
_aggregate = _make_aggregate(False)

BLK = 1000
GRID = N_NODES // BLK


def _update1_body(x_ref, s_ref, c_ref, w_ref, b_ref, h_ref, inv_ref):
  s = s_ref[0] + s_ref[1]
  cnt = c_ref[0, :, 0:1] + c_ref[1, :, 0:1]
  inv = jnp.where(cnt > 0, 1.0 / jnp.maximum(cnt, 1.0), 0.0)
  mean = s * inv
  h = (jnp.dot(x_ref[...], w_ref[0:D, :], preferred_element_type=jnp.float32)
       + jnp.dot(mean, w_ref[D:2 * D, :], preferred_element_type=jnp.float32)
       + b_ref[...])
  h_ref[...] = jnp.maximum(h, 0.0)
  inv_ref[...] = jnp.broadcast_to(inv, (BLK, 16))


def _update2_body(x_ref, s_ref, inv_ref, w_ref, b_ref, o_ref):
  s = s_ref[0] + s_ref[1]
  mean = s * inv_ref[:, 0:1]
  h = (jnp.dot(x_ref[...], w_ref[0:D, :], preferred_element_type=jnp.float32)
       + jnp.dot(mean, w_ref[D:2 * D, :], preferred_element_type=jnp.float32)
       + b_ref[...])
  o_ref[...] = jnp.maximum(h, 0.0)


def _update1(x, s_part, c_part, W, b):
  return pl.pallas_call(
      _update1_body,
      grid=(GRID,),
      in_specs=[
          pl.BlockSpec((BLK, D), lambda i: (i, 0)),
          pl.BlockSpec((NUM_CORES, BLK, D), lambda i: (0, i, 0)),
          pl.BlockSpec((NUM_CORES, BLK, 16), lambda i: (0, i, 0)),
          pl.BlockSpec((2 * D, D), lambda i: (0, 0)),
          pl.BlockSpec((1, D), lambda i: (0, 0)),
      ],
      out_specs=[
          pl.BlockSpec((BLK, D), lambda i: (i, 0)),
          pl.BlockSpec((BLK, 16), lambda i: (i, 0)),
      ],
      out_shape=[
          jax.ShapeDtypeStruct((N_NODES, D), jnp.float32),
          jax.ShapeDtypeStruct((N_NODES, 16), jnp.float32),
      ],
  )(x, s_part, c_part, W, b)


def _update2(x, s_part, inv, W, b):
  return pl.pallas_call(
      _update2_body,
      grid=(GRID,),
      in_specs=[
          pl.BlockSpec((BLK, D), lambda i: (i, 0)),
          pl.BlockSpec((NUM_CORES, BLK, D), lambda i: (0, i, 0)),
          pl.BlockSpec((BLK, 16), lambda i: (i, 0)),
          pl.BlockSpec((2 * D, D), lambda i: (0, 0)),
          pl.BlockSpec((1, D), lambda i: (0, 0)),
      ],
      out_specs=pl.BlockSpec((BLK, D), lambda i: (i, 0)),
      out_shape=jax.ShapeDtypeStruct((N_NODES, D), jnp.float32),
  )(x, s_part, inv, W, b)


def kernel(x, edge_index, W1, b1, W2, b2):
  src = edge_index[0].astype(jnp.int32).reshape(NW, NBLK, BLKC, CHUNK)
  dst = edge_index[1].astype(jnp.int32).reshape(NW, NBLK, BLKC, CHUNK)
  b1r = b1.reshape(1, D)
  b2r = b2.reshape(1, D)
  zrow = jnp.zeros((ROWS_PER_TILE, D), jnp.float32)
  zcnt = jnp.zeros((ROWS_PER_TILE, 16), jnp.float32)
  ones = jnp.ones((CHUNK, 16), jnp.float32)
  s1, c1 = _aggregate_count(x, src, dst, zrow, zcnt, ones)
  h, inv = _update1(x, s1, c1, W1, b1r)
  (s2,) = _aggregate(h, src, dst, zrow, zcnt, ones)
  out = _update2(h, s2, inv, W2, b2r)
  return out
